# baseline (speedup 1.0000x reference)
import functools

import jax
import jax.numpy as jnp
from jax import lax
from jax.experimental import pallas as pl
from jax.experimental.pallas import tpu as pltpu
from jax.experimental.pallas import tpu_sc as plsc

_LLR_CLAMP = 15.0
_TANH_CLAMP = 0.9999

_NC = 2
_NS = 16
_L = 16

_LN2_HI = 0.693359375
_LN2_LO = -2.12194440e-4
_SQRT2 = 1.4142135623730951


def _ln_poly(x):
    bits = lax.bitcast_convert_type(x, jnp.int32)
    e = lax.shift_right_logical(bits, 23) - 127
    mbits = jnp.bitwise_or(
        jnp.bitwise_and(bits, jnp.int32(0x007FFFFF)), jnp.int32(0x3F800000)
    )
    m = lax.bitcast_convert_type(mbits, jnp.float32)
    big = m > _SQRT2
    f = jnp.where(big, m * 0.5, m) - 1.0
    ef = (e + jnp.where(big, jnp.int32(1), jnp.int32(0))).astype(jnp.float32)
    z = f * f
    p = jnp.float32(7.0376836292e-2)
    for c in (-1.1514610310e-1, 1.1676998740e-1, -1.2420140846e-1,
              1.4249322787e-1, -1.6668057665e-1, 2.0000714765e-1,
              -2.4999993993e-1, 3.3333331174e-1):
        p = p * f + jnp.float32(c)
    y = f * z * p
    y = y + _LN2_LO * ef
    y = y - 0.5 * z
    return f + y + _LN2_HI * ef


def _node_tc_body(hE_ref, w_ref, b_ref, la_ref, e2_ref):
    w = w_ref[0, :]
    llr = jnp.sum(hE_ref[...] * w[None, None, :], axis=-1) + b_ref[0, 0]
    llr = jnp.clip(llr, -_LLR_CLAMP, _LLR_CLAMP)
    t = jnp.clip(jnp.tanh(llr * 0.5), -_TANH_CLAMP, _TANH_CLAMP)
    at = jnp.clip(jnp.abs(t), 1e-30, None)
    la_ref[...] = jnp.log(at)
    e2_ref[...] = jnp.sign(t) / at


def _proj_tc_body(m_ref, w_ref, b_ref, out_ref):
    m = m_ref[...]
    out_ref[...] = (m[:, :, None] * w_ref[0, :][None, None, :]
                    + b_ref[0, :][None, None, :])


def _make_edge_kernel(B, NPAD, E, CS, EPT, interpret=False):
    VREGS = EPT // _L
    UA = 5 if VREGS % 5 == 0 else 1
    UC = 5 if VREGS % 5 == 0 else 1
    RW = 128
    assert CS % RW == 0 or CS < RW
    RWW = min(RW, CS)
    NRC = CS // RWW
    CH = min(4000, EPT)
    assert EPT % CH == 0
    NCH = EPT // CH
    CVR = CH // _L

    def body(la_f, e2_f, syn_f, e2d, d2e, out_f,
             idx_a, idx_b, idx_c0, idx_c1, nodeA, nodeB,
             accA, accB, accC, accD, red, ioi, shA, shB, shC, shD, sem):
        b = lax.axis_index("c")
        s = lax.axis_index("s")
        nbase = b * NPAD
        ebase = s * EPT
        off = s * CS

        cps = [
            pltpu.async_copy(e2d.at[pl.ds(ebase, EPT)], idx_a, sem),
            pltpu.async_copy(e2d.at[pl.ds(E + ebase, EPT)], idx_b, sem),
            pltpu.async_copy(la_f.at[pl.ds(nbase, NPAD)], nodeA, sem),
            pltpu.async_copy(e2_f.at[pl.ds(nbase, NPAD)], nodeB, sem),
        ]

        @plsc.parallel_loop(0, NPAD, step=_L, unroll=8)
        def zbody(o):
            zz = jnp.zeros((_L,), jnp.float32)
            accA[pl.ds(o, _L)] = zz
            accB[pl.ds(o, _L)] = zz
            accC[pl.ds(o, _L)] = zz
            accD[pl.ds(o, _L)] = zz
            ioi[pl.ds(o, _L)] = jnp.arange(_L, dtype=jnp.int32) + o

        for sh in (shA, shB, shC, shD):
            pltpu.sync_copy(accA.at[pl.ds(off, CS)], sh.at[pl.ds(off, CS)])

        for c in cps:
            c.wait()

        @plsc.parallel_loop(0, EPT, step=_L, unroll=UA)
        def abody(o):
            si = idx_a[pl.ds(o, _L)]
            di = idx_b[pl.ds(o, _L)]
            la = plsc.load_gather(nodeA, [si])
            e2 = plsc.load_gather(nodeB, [si])
            isneg = jnp.where(e2 < 0.0, 1.0, 0.0).astype(jnp.float32)
            plsc.addupdate_scatter(accA, [di], isneg)
            plsc.addupdate_scatter(accB, [di], la)

        plsc.subcore_barrier()
        pltpu.sync_copy(accA, shA.at[ioi], add=True)
        pltpu.sync_copy(accB, shB.at[ioi], add=True)
        plsc.subcore_barrier()
        pltpu.sync_copy(shA.at[pl.ds(off, CS)], accA.at[pl.ds(off, CS)])
        pltpu.sync_copy(shB.at[pl.ds(off, CS)], accB.at[pl.ds(off, CS)])

        for c in range(NRC):
            pltpu.sync_copy(
                syn_f.at[pl.ds(nbase + off + c * RWW, RWW)], red.at[0])

            def cbody(i, _):
                o = i * _L
                ncnt = accA[pl.ds(off + c * RWW + o, _L)]
                lat = accB[pl.ds(off + c * RWW + o, _L)]
                syn = red[0, pl.ds(o, _L)]
                par = jnp.bitwise_and(ncnt.astype(jnp.int32),
                                      jnp.int32(1)).astype(jnp.float32)
                sps = (1.0 - 2.0 * par) * (1.0 - 2.0 * syn)
                accA[pl.ds(off + c * RWW + o, _L)] = sps * jnp.exp(lat)
                return 0
            lax.fori_loop(0, RWW // _L, cbody, 0)

        pltpu.sync_copy(accA.at[pl.ds(off, CS)], shA.at[pl.ds(off, CS)])
        cpe = pltpu.async_copy(d2e.at[pl.ds(E + ebase, CH)], idx_c0, sem)
        plsc.subcore_barrier()
        pltpu.sync_copy(shA, accA)

        cbufs = [idx_c0, idx_c1]
        cpe.wait()
        for k in range(NCH):
            cur = cbufs[k % 2]
            if k + 1 < NCH:
                nxt_cp = pltpu.async_copy(
                    d2e.at[pl.ds(E + ebase + (k + 1) * CH, CH)],
                    cbufs[(k + 1) % 2], sem)

            @plsc.parallel_loop(0, CH, step=_L, unroll=UC)
            def ebody(oc):
                o = k * CH + oc
                si = idx_a[pl.ds(o, _L)]
                di = idx_b[pl.ds(o, _L)]
                ei = cur[pl.ds(oc, _L)]
                e2 = plsc.load_gather(nodeB, [si])
                comb = plsc.load_gather(accA, [di])
                et = jnp.clip(e2 * comb, -_TANH_CLAMP, _TANH_CLAMP)
                msg = _ln_poly((1.0 + et) / (1.0 - et))
                plsc.addupdate_scatter(accC, [ei], msg)
                plsc.addupdate_scatter(accD, [ei],
                                       jnp.ones((_L,), jnp.float32))

            if k + 1 < NCH:
                nxt_cp.wait()

        pltpu.sync_copy(accC, shC.at[ioi], add=True)
        pltpu.sync_copy(accD, shD.at[ioi], add=True)
        plsc.subcore_barrier()
        pltpu.sync_copy(shC.at[pl.ds(off, CS)], accC.at[pl.ds(off, CS)])
        pltpu.sync_copy(shD.at[pl.ds(off, CS)], accD.at[pl.ds(off, CS)])

        def dbody(i, _):
            o = i * _L
            bp = accC[pl.ds(off + o, _L)]
            cnt = accD[pl.ds(off + o, _L)]
            accC[pl.ds(off + o, _L)] = bp / jnp.maximum(cnt, 1.0)
            return 0
        lax.fori_loop(0, CS // _L, dbody, 0)

        pltpu.sync_copy(accC.at[pl.ds(off, CS)],
                        out_f.at[pl.ds(nbase + off, CS)])

    return pl.kernel(
        body,
        out_type=jax.ShapeDtypeStruct((B * NPAD,), jnp.float32),
        mesh=plsc.VectorSubcoreMesh(core_axis_name="c", subcore_axis_name="s",
                                    num_cores=_NC, num_subcores=_NS),
        compiler_params=pltpu.CompilerParams(needs_layout_passes=False),
        scratch_types=[
            pltpu.VMEM((EPT,), jnp.int32),
            pltpu.VMEM((EPT,), jnp.int32),
            pltpu.VMEM((CH,), jnp.int32),
            pltpu.VMEM((CH,), jnp.int32),
            pltpu.VMEM((NPAD,), jnp.float32),
            pltpu.VMEM((NPAD,), jnp.float32),
            pltpu.VMEM((NPAD,), jnp.float32),
            pltpu.VMEM((NPAD,), jnp.float32),
            pltpu.VMEM((NPAD,), jnp.float32),
            pltpu.VMEM((NPAD,), jnp.float32),
            pltpu.VMEM((_NS, min(128, CS)), jnp.float32),
            pltpu.VMEM((NPAD,), jnp.int32),
            pltpu.VMEM_SHARED((NPAD,), jnp.float32),
            pltpu.VMEM_SHARED((NPAD,), jnp.float32),
            pltpu.VMEM_SHARED((NPAD,), jnp.float32),
            pltpu.VMEM_SHARED((NPAD,), jnp.float32),
            pltpu.SemaphoreType.DMA,
        ],
        interpret=interpret,
    )


@functools.cache
def _make_pipeline(B, N, H, E, interpret=False):
    assert B == _NC, "core axis carries the batch"
    assert E % _NS == 0
    EPT = E // _NS
    assert EPT % _L == 0
    NPAD = -(-N // (_NS * _L)) * (_NS * _L)
    CS = NPAD // _NS

    edge_call = _make_edge_kernel(B, NPAD, E, CS, EPT, interpret=interpret)

    NB = 1024 if NPAD % 1024 == 0 else NPAD
    NG = NPAD // NB

    node_call = pl.pallas_call(
        _node_tc_body,
        grid=(NG,),
        in_specs=[pl.BlockSpec((B, NB, H), lambda i: (0, i, 0)),
                  pl.BlockSpec((1, H), lambda i: (0, 0)),
                  pl.BlockSpec((1, 1), lambda i: (0, 0))],
        out_specs=[pl.BlockSpec((B, NB), lambda i: (0, i)),
                   pl.BlockSpec((B, NB), lambda i: (0, i))],
        out_shape=[jax.ShapeDtypeStruct((B, NPAD), jnp.float32),
                   jax.ShapeDtypeStruct((B, NPAD), jnp.float32)],
        interpret=interpret,
    )

    proj_call = pl.pallas_call(
        _proj_tc_body,
        grid=(NG,),
        in_specs=[pl.BlockSpec((B, NB), lambda i: (0, i)),
                  pl.BlockSpec((1, H), lambda i: (0, 0)),
                  pl.BlockSpec((1, H), lambda i: (0, 0))],
        out_specs=pl.BlockSpec((B, NB, H), lambda i: (0, i, 0)),
        out_shape=jax.ShapeDtypeStruct((B, N, H), jnp.float32),
        interpret=interpret,
    )

    def run(hE, syndrome, e2d, d2e, W_llr, b_llr, W_msg, b_msg):
        la, sg = node_call(hE, W_llr, b_llr.reshape(1, 1))
        la_f = la.reshape(-1)
        sg_f = sg.reshape(-1)
        syn_f = jnp.pad(syndrome, ((0, 0), (0, NPAD - N))).reshape(-1)
        mean_f = edge_call(la_f, sg_f, syn_f,
                           e2d.reshape(-1), d2e.reshape(-1))
        mean = mean_f.reshape(B, NPAD)
        return proj_call(mean, W_msg.reshape(1, H), b_msg.reshape(1, H))

    return run


def kernel(hE, hD, syndrome, edge_e2d, edge_d2e, W_llr, b_llr, W_msg, b_msg):
    B, N, H = hE.shape
    E = edge_e2d.shape[1]
    run = _make_pipeline(B, N, H, E)
    return run(hE, syndrome.astype(jnp.float32),
               edge_e2d.astype(jnp.int32), edge_d2e.astype(jnp.int32),
               W_llr, b_llr, W_msg, b_msg)


# speedup vs baseline: 1.0373x; 1.0373x over previous
import functools

import jax
import jax.numpy as jnp
from jax import lax
from jax.experimental import pallas as pl
from jax.experimental.pallas import tpu as pltpu
from jax.experimental.pallas import tpu_sc as plsc

_LLR_CLAMP = 15.0
_TANH_CLAMP = 0.9999

_NC = 2
_NS = 16
_L = 16

_LN2_HI = 0.693359375
_LN2_LO = -2.12194440e-4
_SQRT2 = 1.4142135623730951


def _ln_poly(x):
    bits = lax.bitcast_convert_type(x, jnp.int32)
    e = lax.shift_right_logical(bits, 23) - 127
    mbits = jnp.bitwise_or(
        jnp.bitwise_and(bits, jnp.int32(0x007FFFFF)), jnp.int32(0x3F800000)
    )
    m = lax.bitcast_convert_type(mbits, jnp.float32)
    big = m > _SQRT2
    f = jnp.where(big, m * 0.5, m) - 1.0
    ef = (e + jnp.where(big, jnp.int32(1), jnp.int32(0))).astype(jnp.float32)
    z = f * f
    p = jnp.float32(-1.2420140846e-1)
    for c in (1.4249322787e-1, -1.6668057665e-1, 2.0000714765e-1,
              -2.4999993993e-1, 3.3333331174e-1):
        p = p * f + jnp.float32(c)
    y = f * z * p
    y = y + _LN2_LO * ef
    y = y - 0.5 * z
    return f + y + _LN2_HI * ef


def _node_tc_body(hE_ref, w_ref, b_ref, la_ref, e2_ref):
    w = w_ref[0, :]
    llr = jnp.sum(hE_ref[...] * w[None, None, :], axis=-1) + b_ref[0, 0]
    llr = jnp.clip(llr, -_LLR_CLAMP, _LLR_CLAMP)
    t = jnp.clip(jnp.tanh(llr * 0.5), -_TANH_CLAMP, _TANH_CLAMP)
    at = jnp.clip(jnp.abs(t), 1e-30, None)
    la_ref[...] = jnp.log(at)
    e2_ref[...] = jnp.sign(t) / at


def _proj_tc_body(m_ref, w_ref, b_ref, out_ref):
    m = m_ref[...]
    out_ref[...] = (m[:, :, None] * w_ref[0, :][None, None, :]
                    + b_ref[0, :][None, None, :])


def _make_edge_kernel(B, NPAD, E, CS, EPT, interpret=False):
    VREGS = EPT // _L
    UA = 5 if VREGS % 5 == 0 else 1
    UC = 5 if VREGS % 5 == 0 else 1
    RW = 128
    assert CS % RW == 0 or CS < RW
    RWW = min(RW, CS)
    NRC = CS // RWW
    CH = min(4000, EPT)
    assert EPT % CH == 0
    NCH = EPT // CH
    CVR = CH // _L

    def _reduce_slice(sh, dst, off, red):
        for c in range(NRC):
            pltpu.sync_copy(sh.at[:, pl.ds(off + c * RWW, RWW)], red)

            def rbody(i, _):
                o = i * _L
                v = red[0, pl.ds(o, _L)]
                for r in range(1, _NS):
                    v = v + red[r, pl.ds(o, _L)]
                dst[pl.ds(off + c * RWW + o, _L)] = v
                return 0
            lax.fori_loop(0, RWW // _L, rbody, 0)

    def body(la_f, e2_f, syn_f, e2d, d2e, out_f,
             idx_a, idx_b, idx_c0, idx_c1, nodeA, nodeB,
             accA, accB, accC, accD, red, shP, shT, sem):
        b = lax.axis_index("c")
        s = lax.axis_index("s")
        nbase = b * NPAD
        ebase = s * EPT
        off = s * CS

        cps = [
            pltpu.async_copy(e2d.at[pl.ds(ebase, EPT)], idx_a, sem),
            pltpu.async_copy(e2d.at[pl.ds(E + ebase, EPT)], idx_b, sem),
            pltpu.async_copy(la_f.at[pl.ds(nbase, NPAD)], nodeA, sem),
            pltpu.async_copy(e2_f.at[pl.ds(nbase, NPAD)], nodeB, sem),
        ]

        @plsc.parallel_loop(0, NPAD, step=_L, unroll=8)
        def zbody(o):
            zz = jnp.zeros((_L,), jnp.float32)
            accA[pl.ds(o, _L)] = zz
            accB[pl.ds(o, _L)] = zz
            accC[pl.ds(o, _L)] = zz
            accD[pl.ds(o, _L)] = zz

        for c in cps:
            c.wait()

        @plsc.parallel_loop(0, EPT, step=_L, unroll=UA)
        def abody(o):
            si = idx_a[pl.ds(o, _L)]
            di = idx_b[pl.ds(o, _L)]
            la = plsc.load_gather(nodeA, [si])
            e2 = plsc.load_gather(nodeB, [si])
            isneg = jnp.where(e2 < 0.0, 1.0, 0.0).astype(jnp.float32)
            plsc.addupdate_scatter(accA, [di], isneg)
            plsc.addupdate_scatter(accB, [di], la)

        pltpu.sync_copy(accA, shP.at[s])
        plsc.subcore_barrier()
        _reduce_slice(shP, accA, off, red)
        plsc.subcore_barrier()
        pltpu.sync_copy(accB, shP.at[s])
        plsc.subcore_barrier()
        _reduce_slice(shP, accB, off, red)

        for c in range(NRC):
            pltpu.sync_copy(
                syn_f.at[pl.ds(nbase + off + c * RWW, RWW)], red.at[0])

            def cbody(i, _):
                o = i * _L
                ncnt = accA[pl.ds(off + c * RWW + o, _L)]
                lat = accB[pl.ds(off + c * RWW + o, _L)]
                syn = red[0, pl.ds(o, _L)]
                par = jnp.bitwise_and(ncnt.astype(jnp.int32),
                                      jnp.int32(1)).astype(jnp.float32)
                sps = (1.0 - 2.0 * par) * (1.0 - 2.0 * syn)
                accA[pl.ds(off + c * RWW + o, _L)] = sps * jnp.exp(lat)
                return 0
            lax.fori_loop(0, RWW // _L, cbody, 0)

        pltpu.sync_copy(accA.at[pl.ds(off, CS)], shT.at[pl.ds(off, CS)])
        cpe = pltpu.async_copy(d2e.at[pl.ds(E + ebase, CH)], idx_c0, sem)
        plsc.subcore_barrier()
        pltpu.sync_copy(shT, accA)

        cbufs = [idx_c0, idx_c1]
        cpe.wait()
        for k in range(NCH):
            cur = cbufs[k % 2]
            if k + 1 < NCH:
                nxt_cp = pltpu.async_copy(
                    d2e.at[pl.ds(E + ebase + (k + 1) * CH, CH)],
                    cbufs[(k + 1) % 2], sem)

            @plsc.parallel_loop(0, CH, step=_L, unroll=UC)
            def ebody(oc):
                o = k * CH + oc
                si = idx_a[pl.ds(o, _L)]
                di = idx_b[pl.ds(o, _L)]
                ei = cur[pl.ds(oc, _L)]
                e2 = plsc.load_gather(nodeB, [si])
                comb = plsc.load_gather(accA, [di])
                et = jnp.clip(e2 * comb, -_TANH_CLAMP, _TANH_CLAMP)
                msg = _ln_poly((1.0 + et) / (1.0 - et))
                plsc.addupdate_scatter(accC, [ei], msg)
                plsc.addupdate_scatter(accD, [ei],
                                       jnp.ones((_L,), jnp.float32))

            if k + 1 < NCH:
                nxt_cp.wait()

        pltpu.sync_copy(accC, shP.at[s])
        plsc.subcore_barrier()
        _reduce_slice(shP, accC, off, red)
        plsc.subcore_barrier()
        pltpu.sync_copy(accD, shP.at[s])
        plsc.subcore_barrier()
        _reduce_slice(shP, accD, off, red)

        def dbody(i, _):
            o = i * _L
            bp = accC[pl.ds(off + o, _L)]
            cnt = accD[pl.ds(off + o, _L)]
            accC[pl.ds(off + o, _L)] = bp / jnp.maximum(cnt, 1.0)
            return 0
        lax.fori_loop(0, CS // _L, dbody, 0)

        pltpu.sync_copy(accC.at[pl.ds(off, CS)],
                        out_f.at[pl.ds(nbase + off, CS)])

    return pl.kernel(
        body,
        out_type=jax.ShapeDtypeStruct((B * NPAD,), jnp.float32),
        mesh=plsc.VectorSubcoreMesh(core_axis_name="c", subcore_axis_name="s",
                                    num_cores=_NC, num_subcores=_NS),
        compiler_params=pltpu.CompilerParams(needs_layout_passes=False),
        scratch_types=[
            pltpu.VMEM((EPT,), jnp.int32),
            pltpu.VMEM((EPT,), jnp.int32),
            pltpu.VMEM((CH,), jnp.int32),
            pltpu.VMEM((CH,), jnp.int32),
            pltpu.VMEM((NPAD,), jnp.float32),
            pltpu.VMEM((NPAD,), jnp.float32),
            pltpu.VMEM((NPAD,), jnp.float32),
            pltpu.VMEM((NPAD,), jnp.float32),
            pltpu.VMEM((NPAD,), jnp.float32),
            pltpu.VMEM((NPAD,), jnp.float32),
            pltpu.VMEM((_NS, min(128, CS)), jnp.float32),
            pltpu.VMEM_SHARED((_NS, NPAD), jnp.float32),
            pltpu.VMEM_SHARED((NPAD,), jnp.float32),
            pltpu.SemaphoreType.DMA,
        ],
        interpret=interpret,
    )


@functools.cache
def _make_pipeline(B, N, H, E, interpret=False):
    assert B == _NC, "core axis carries the batch"
    assert E % _NS == 0
    EPT = E // _NS
    assert EPT % _L == 0
    NPAD = -(-N // (_NS * _L)) * (_NS * _L)
    CS = NPAD // _NS

    edge_call = _make_edge_kernel(B, NPAD, E, CS, EPT, interpret=interpret)

    NB = 1024 if NPAD % 1024 == 0 else NPAD
    NG = NPAD // NB

    node_call = pl.pallas_call(
        _node_tc_body,
        grid=(NG,),
        in_specs=[pl.BlockSpec((B, NB, H), lambda i: (0, i, 0)),
                  pl.BlockSpec((1, H), lambda i: (0, 0)),
                  pl.BlockSpec((1, 1), lambda i: (0, 0))],
        out_specs=[pl.BlockSpec((B, NB), lambda i: (0, i)),
                   pl.BlockSpec((B, NB), lambda i: (0, i))],
        out_shape=[jax.ShapeDtypeStruct((B, NPAD), jnp.float32),
                   jax.ShapeDtypeStruct((B, NPAD), jnp.float32)],
        interpret=interpret,
    )

    proj_call = pl.pallas_call(
        _proj_tc_body,
        grid=(NG,),
        in_specs=[pl.BlockSpec((B, NB), lambda i: (0, i)),
                  pl.BlockSpec((1, H), lambda i: (0, 0)),
                  pl.BlockSpec((1, H), lambda i: (0, 0))],
        out_specs=pl.BlockSpec((B, NB, H), lambda i: (0, i, 0)),
        out_shape=jax.ShapeDtypeStruct((B, N, H), jnp.float32),
        interpret=interpret,
    )

    def run(hE, syndrome, e2d, d2e, W_llr, b_llr, W_msg, b_msg):
        la, sg = node_call(hE, W_llr, b_llr.reshape(1, 1))
        la_f = la.reshape(-1)
        sg_f = sg.reshape(-1)
        syn_f = jnp.pad(syndrome, ((0, 0), (0, NPAD - N))).reshape(-1)
        mean_f = edge_call(la_f, sg_f, syn_f,
                           e2d.reshape(-1), d2e.reshape(-1))
        mean = mean_f.reshape(B, NPAD)
        return proj_call(mean, W_msg.reshape(1, H), b_msg.reshape(1, H))

    return run


def kernel(hE, hD, syndrome, edge_e2d, edge_d2e, W_llr, b_llr, W_msg, b_msg):
    B, N, H = hE.shape
    E = edge_e2d.shape[1]
    run = _make_pipeline(B, N, H, E)
    return run(hE, syndrome.astype(jnp.float32),
               edge_e2d.astype(jnp.int32), edge_d2e.astype(jnp.int32),
               W_llr, b_llr, W_msg, b_msg)
